# Initial kernel scaffold; baseline (speedup 1.0000x reference)
import jax
import jax.numpy as jnp
from jax.experimental import pallas as pl


def kernel(x, Wr, br, W1, b1, W2, b2):
    raise NotImplementedError("write your pallas kernel here")


# speedup vs baseline: 1.2830x; 1.2830x over previous
import functools

import jax
import jax.numpy as jnp
from jax.experimental import pallas as pl
from jax.experimental.pallas import tpu as pltpu

EMBED = 1024
HIDDEN = 4096
NUM_EXPERTS = 8
T = 2048
HB = 1024
N_HB = HIDDEN // HB


def _moe_body(x_ref, wr_ref, br_ref, b2_ref, w1_ref, b1_ref, w2_ref,
              out_ref, probs_ref):
    n = pl.program_id(0)
    hb = pl.program_id(1)

    @pl.when((n == 0) & (hb == 0))
    def _init():
        logits = jnp.dot(x_ref[...], wr_ref[...].astype(jnp.bfloat16),
                         preferred_element_type=jnp.float32) + br_ref[...]
        m = jnp.max(logits, axis=-1, keepdims=True)
        e = jnp.exp(logits - m)
        p = e / jnp.sum(e, axis=-1, keepdims=True)
        probs_ref[...] = p
        out_ref[...] = jnp.dot(p, b2_ref[...],
                               preferred_element_type=jnp.float32)

    h = jnp.dot(x_ref[...], w1_ref[0].astype(jnp.bfloat16),
                preferred_element_type=jnp.float32)
    h = jnp.maximum(h + b1_ref[0], 0)
    lane = jax.lax.broadcasted_iota(jnp.int32, (T, NUM_EXPERTS), 1)
    p_col = jnp.sum(jnp.where(lane == n, probs_ref[...], 0.0), axis=1,
                    keepdims=True)
    hp = (h * p_col).astype(jnp.bfloat16)
    out_ref[...] += jnp.dot(hp, w2_ref[0].astype(jnp.bfloat16),
                            preferred_element_type=jnp.float32)


@functools.partial(jax.jit, static_argnames=())
def kernel(x, Wr, br, W1, b1, W2, b2):
    b, t, d = x.shape
    xb = x.reshape(t, d).astype(jnp.bfloat16)
    out = pl.pallas_call(
        _moe_body,
        grid=(NUM_EXPERTS, N_HB),
        in_specs=[
            pl.BlockSpec((T, EMBED), lambda n, h: (0, 0)),
            pl.BlockSpec((EMBED, NUM_EXPERTS), lambda n, h: (0, 0)),
            pl.BlockSpec((1, NUM_EXPERTS), lambda n, h: (0, 0)),
            pl.BlockSpec((NUM_EXPERTS, EMBED), lambda n, h: (0, 0)),
            pl.BlockSpec((1, EMBED, HB), lambda n, h: (n, 0, h)),
            pl.BlockSpec((1, 1, HB), lambda n, h: (n * N_HB + h, 0, 0)),
            pl.BlockSpec((1, HB, EMBED), lambda n, h: (n, h, 0)),
        ],
        out_specs=pl.BlockSpec((T, EMBED), lambda n, h: (0, 0)),
        out_shape=jax.ShapeDtypeStruct((T, EMBED), jnp.float32),
        scratch_shapes=[pltpu.VMEM((T, NUM_EXPERTS), jnp.float32)],
    )(xb, Wr, br.reshape(1, NUM_EXPERTS), b2, W1,
      b1.reshape(NUM_EXPERTS * N_HB, 1, HB), W2)
    return out.reshape(b, t, d)


# speedup vs baseline: 1.2858x; 1.0022x over previous
import functools

import jax
import jax.numpy as jnp
print("DEVCHECK:", jax.devices(), jax.local_device_count())
from jax.experimental import pallas as pl
from jax.experimental.pallas import tpu as pltpu

EMBED = 1024
HIDDEN = 4096
NUM_EXPERTS = 8
T = 2048
HB = 1024
N_HB = HIDDEN // HB


def _moe_body(x_ref, wr_ref, br_ref, b2_ref, w1_ref, b1_ref, w2_ref,
              out_ref, probs_ref):
    n = pl.program_id(0)
    hb = pl.program_id(1)

    @pl.when((n == 0) & (hb == 0))
    def _init():
        logits = jnp.dot(x_ref[...], wr_ref[...].astype(jnp.bfloat16),
                         preferred_element_type=jnp.float32) + br_ref[...]
        m = jnp.max(logits, axis=-1, keepdims=True)
        e = jnp.exp(logits - m)
        p = e / jnp.sum(e, axis=-1, keepdims=True)
        probs_ref[...] = p
        out_ref[...] = jnp.dot(p, b2_ref[...],
                               preferred_element_type=jnp.float32)

    h = jnp.dot(x_ref[...], w1_ref[0].astype(jnp.bfloat16),
                preferred_element_type=jnp.float32)
    h = jnp.maximum(h + b1_ref[0], 0)
    lane = jax.lax.broadcasted_iota(jnp.int32, (T, NUM_EXPERTS), 1)
    p_col = jnp.sum(jnp.where(lane == n, probs_ref[...], 0.0), axis=1,
                    keepdims=True)
    hp = (h * p_col).astype(jnp.bfloat16)
    out_ref[...] += jnp.dot(hp, w2_ref[0].astype(jnp.bfloat16),
                            preferred_element_type=jnp.float32)


@functools.partial(jax.jit, static_argnames=())
def kernel(x, Wr, br, W1, b1, W2, b2):
    b, t, d = x.shape
    xb = x.reshape(t, d).astype(jnp.bfloat16)
    out = pl.pallas_call(
        _moe_body,
        grid=(NUM_EXPERTS, N_HB),
        in_specs=[
            pl.BlockSpec((T, EMBED), lambda n, h: (0, 0)),
            pl.BlockSpec((EMBED, NUM_EXPERTS), lambda n, h: (0, 0)),
            pl.BlockSpec((1, NUM_EXPERTS), lambda n, h: (0, 0)),
            pl.BlockSpec((NUM_EXPERTS, EMBED), lambda n, h: (0, 0)),
            pl.BlockSpec((1, EMBED, HB), lambda n, h: (n, 0, h)),
            pl.BlockSpec((1, 1, HB), lambda n, h: (n * N_HB + h, 0, 0)),
            pl.BlockSpec((1, HB, EMBED), lambda n, h: (n, h, 0)),
        ],
        out_specs=pl.BlockSpec((T, EMBED), lambda n, h: (0, 0)),
        out_shape=jax.ShapeDtypeStruct((T, EMBED), jnp.float32),
        scratch_shapes=[pltpu.VMEM((T, NUM_EXPERTS), jnp.float32)],
    )(xb, Wr, br.reshape(1, NUM_EXPERTS), b2, W1,
      b1.reshape(NUM_EXPERTS * N_HB, 1, HB), W2)
    return out.reshape(b, t, d)
